# baseline (speedup 1.0000x reference)
import functools

import jax
import jax.numpy as jnp
from jax import lax
from jax.experimental import pallas as pl
from jax.experimental.pallas import tpu as pltpu
from jax.experimental.pallas import tpu_sc as plsc

_B_ROWS = 4096
_SEQ = 50
_D = 64
_NC = 2
_NS = 16
_NW = _NC * _NS
_C = _B_ROWS // _NW
_NBUF = 5
_NT = _SEQ // _NBUF

_mesh = plsc.VectorSubcoreMesh(core_axis_name="c", subcore_axis_name="s")


@functools.partial(
    pl.kernel,
    mesh=_mesh,
    out_type=jax.ShapeDtypeStruct((_SEQ, _D // 8, _NW, 8 * _C), jnp.float32),
    scratch_types=[
        pltpu.VMEM((_SEQ, _C), jnp.int32),
        pltpu.VMEM((_NBUF, _C, _D), jnp.float32),
        pltpu.VMEM((_NBUF, _D * _C), jnp.float32),
        pltpu.SemaphoreType.DMA((_NBUF,)),
        pltpu.SemaphoreType.DMA((_NBUF,)),
    ],
    compiler_params=pltpu.CompilerParams(
        use_tc_tiling_on_sc=False, needs_layout_passes=False
    ),
)
def _sc_gather(idx_hbm, table_hbm, out_hbm, idx_v, rows_v, trans_v, gsem, wsem):
    wid = lax.axis_index("s") * _NC + lax.axis_index("c")
    pltpu.sync_copy(idx_hbm.at[:, pl.ds(wid * _C, _C)], idx_v)

    iota = lax.iota(jnp.int32, 16)

    def gather_start(b, s):
        pltpu.make_async_copy(
            table_hbm.at[idx_v.at[s]], rows_v.at[b], gsem.at[b]
        ).start()

    def gather_wait(b):
        pltpu.make_async_copy(
            table_hbm.at[idx_v.at[0]], rows_v.at[b], gsem.at[b]
        ).wait()

    def write_start(b, s):
        for tr in range(_D // 8):
            pltpu.make_async_copy(
                trans_v.at[b, pl.ds(tr * 8 * _C, 8 * _C)],
                out_hbm.at[s, tr, wid],
                wsem.at[b],
            ).start()

    def write_wait(b):
        for tr in range(_D // 8):
            pltpu.make_async_copy(
                trans_v.at[b, pl.ds(tr * 8 * _C, 8 * _C)],
                out_hbm.at[0, tr, wid],
                wsem.at[b],
            ).wait()

    scat = [((iota + 16 * k) >> 3) * (8 * _C) + ((iota + 16 * k) & 7) * _C
            for k in range(_D // 16)]

    def transpose_chunk(b):
        trans_b = trans_v.at[b]

        @plsc.parallel_loop(0, _C, unroll=4)
        def cbody(c):
            cvec = jnp.full((16,), 0, jnp.int32) + c
            for k in range(_D // 16):
                vals = rows_v[b, c, pl.ds(16 * k, 16)]
                plsc.store_scatter(trans_b, [scat[k] + cvec], vals)

    for b in range(_NBUF):
        gather_start(b, b)

    def step(t, carry):
        for b in range(_NBUF):
            s = t * _NBUF + b
            gather_wait(b)

            @pl.when(t >= 1)
            def _():
                write_wait(b)

            transpose_chunk(b)
            write_start(b, s)

            @pl.when(t < _NT - 1)
            def _():
                gather_start(b, s + _NBUF)

        return carry

    lax.fori_loop(0, _NT, step, 0)

    for b in range(_NBUF):
        write_wait(b)


def kernel(inputs, embedding):
    idx = inputs.astype(jnp.int32).T
    out = _sc_gather(idx, embedding)
    out = out.reshape(_SEQ, _D // 8, _NW, 8, _C)
    out = out.transpose(2, 4, 0, 1, 3).reshape(_B_ROWS, _SEQ, _D)
    return out


# speedup vs baseline: 2.2044x; 2.2044x over previous
import functools

import jax
import jax.numpy as jnp
from jax import lax
from jax.experimental import pallas as pl
from jax.experimental.pallas import tpu as pltpu
from jax.experimental.pallas import tpu_sc as plsc

_B_ROWS = 4096
_SEQ = 50
_D = 64
_NC = 2
_NS = 16
_NW = _NC * _NS
_C = _B_ROWS // _NW
_NBUF = 5
_NT = _SEQ // _NBUF

_mesh = plsc.VectorSubcoreMesh(core_axis_name="c", subcore_axis_name="s")


@functools.partial(
    pl.kernel,
    mesh=_mesh,
    out_type=jax.ShapeDtypeStruct((_SEQ, _D // 8, _NW, 8, _C), jnp.float32),
    scratch_types=[
        pltpu.VMEM((_SEQ, _C), jnp.int32),
        pltpu.VMEM((_NBUF, _C, _D), jnp.float32),
        pltpu.VMEM((_NBUF, _D, _C + 1), jnp.float32),
        pltpu.SemaphoreType.DMA((_NBUF,)),
        pltpu.SemaphoreType.DMA((_NBUF,)),
    ],
    compiler_params=pltpu.CompilerParams(
        use_tc_tiling_on_sc=False, needs_layout_passes=False
    ),
)
def _sc_gather(idx_hbm, table_hbm, out_hbm, idx_v, rows_v, trans_v, gsem, wsem):
    wid = lax.axis_index("s") * _NC + lax.axis_index("c")
    pltpu.sync_copy(idx_hbm.at[:, pl.ds(wid * _C, _C)], idx_v)

    iota = lax.iota(jnp.int32, 16)

    def gather_start(b, s):
        pltpu.make_async_copy(
            table_hbm.at[idx_v.at[s]], rows_v.at[b], gsem.at[b]
        ).start()

    def gather_wait(b):
        pltpu.make_async_copy(
            table_hbm.at[idx_v.at[0]], rows_v.at[b], gsem.at[b]
        ).wait()

    def write_start(b, s):
        for tr in range(_D // 8):
            pltpu.make_async_copy(
                trans_v.at[b, pl.ds(tr * 8, 8), pl.ds(0, _C)],
                out_hbm.at[s, tr, wid],
                wsem.at[b],
            ).start()

    def write_wait(b):
        for tr in range(_D // 8):
            pltpu.make_async_copy(
                trans_v.at[b, pl.ds(tr * 8, 8), pl.ds(0, _C)],
                out_hbm.at[0, tr, wid],
                wsem.at[b],
            ).wait()

    dvecs = [iota + 16 * k for k in range(_D // 16)]

    def transpose_chunk(b):
        trans_b = trans_v.at[b]

        @plsc.parallel_loop(0, _C, unroll=4)
        def cbody(c):
            cvec = jnp.full((16,), 0, jnp.int32) + c
            for k in range(_D // 16):
                vals = rows_v[b, c, pl.ds(16 * k, 16)]
                plsc.store_scatter(trans_b, [dvecs[k], cvec], vals)

    for b in range(_NBUF):
        gather_start(b, b)

    def step(t, carry):
        for b in range(_NBUF):
            s = t * _NBUF + b
            gather_wait(b)

            @pl.when(t >= 1)
            def _():
                write_wait(b)

            transpose_chunk(b)
            write_start(b, s)

            @pl.when(t < _NT - 1)
            def _():
                gather_start(b, s + _NBUF)

        return carry

    lax.fori_loop(0, _NT, step, 0)

    for b in range(_NBUF):
        write_wait(b)


def kernel(inputs, embedding):
    idx = inputs.astype(jnp.int32).T
    out = _sc_gather(idx, embedding)
    out = out.transpose(2, 4, 0, 1, 3).reshape(_B_ROWS, _SEQ, _D)
    return out
